# baseline (speedup 1.0000x reference)
import functools

import numpy as np

import jax
import jax.numpy as jnp
from jax import lax
from jax.experimental import pallas as pl
from jax.experimental.pallas import tpu as pltpu
from jax.experimental.pallas import tpu_sc as plsc

N = 10000
E = 320000
D = 128
DE = 16

NC = 2
NS = 16
NW = NC * NS

KCH = 2
EPC = E // KCH
CH = 40

ZCH = 80
NZCH = N // ZCH
KMAX = (NZCH + NS - 1) // NS

BE = 8000


def _gather_diff_body(epw, nchunk,
                      state_hbm, src_hbm, dst_hbm, diff_hbm,
                      sidx_all, didx_all,
                      srows0, srows1, drows0, drows1, dbuf0, dbuf1,
                      gs0, gs1, gd0, gd1, wb0, wb1):
    wid = lax.axis_index("s") * NC + lax.axis_index("c")
    base = wid * epw
    srows = (srows0, srows1)
    drows = (drows0, drows1)
    dbuf = (dbuf0, dbuf1)
    gs = (gs0, gs1)
    gd = (gd0, gd1)
    wb = (wb0, wb1)

    pltpu.sync_copy(src_hbm.at[pl.ds(base, epw)], sidx_all)
    pltpu.sync_copy(dst_hbm.at[pl.ds(base, epw)], didx_all)

    def issue_gather(j, b):
        pltpu.async_copy(
            state_hbm.at[sidx_all.at[pl.ds(j * CH, CH)]], srows[b], gs[b])
        pltpu.async_copy(
            state_hbm.at[didx_all.at[pl.ds(j * CH, CH)]], drows[b], gd[b])

    issue_gather(0, 0)
    issue_gather(1, 1)

    def pair(g, c):
        for b in range(2):
            j = 2 * g + b

            @pl.when(j < nchunk)
            def _():
                off = base + j * CH
                pltpu.make_async_copy(
                    state_hbm.at[sidx_all.at[pl.ds(j * CH, CH)]],
                    srows[b], gs[b]).wait()
                pltpu.make_async_copy(
                    state_hbm.at[didx_all.at[pl.ds(j * CH, CH)]],
                    drows[b], gd[b]).wait()

                @pl.when(j >= 2)
                def _():
                    pltpu.make_async_copy(
                        dbuf[b], diff_hbm.at[pl.ds(off, CH)], wb[b]).wait()

                @plsc.parallel_loop(0, CH, 1, unroll=4)
                def _(r):
                    for col in range(D // 16):
                        sl = (r, pl.ds(col * 16, 16))
                        dbuf[b][sl] = srows[b][sl] - drows[b][sl]

                pltpu.async_copy(dbuf[b], diff_hbm.at[pl.ds(off, CH)], wb[b])

                @pl.when(j + 2 < nchunk)
                def _():
                    issue_gather(j + 2, b)
        return c

    lax.fori_loop(0, (nchunk + 1) // 2, pair, 0)
    for b in range(2):
        pltpu.make_async_copy(
            dbuf[b], diff_hbm.at[pl.ds(base, CH)], wb[b]).wait()


def _sc_gather_diff(state, src, dst, ne):
    epw = ne // NW
    nchunk = epw // CH
    mesh = plsc.VectorSubcoreMesh(core_axis_name="c", subcore_axis_name="s")
    f = functools.partial(
        pl.kernel,
        out_type=jax.ShapeDtypeStruct((ne, D), jnp.float32),
        mesh=mesh,
        scratch_types=[
            pltpu.VMEM((epw,), jnp.int32),
            pltpu.VMEM((epw,), jnp.int32),
            pltpu.VMEM((CH, D), jnp.float32),
            pltpu.VMEM((CH, D), jnp.float32),
            pltpu.VMEM((CH, D), jnp.float32),
            pltpu.VMEM((CH, D), jnp.float32),
            pltpu.VMEM((CH, D), jnp.float32),
            pltpu.VMEM((CH, D), jnp.float32),
            pltpu.SemaphoreType.DMA,
            pltpu.SemaphoreType.DMA,
            pltpu.SemaphoreType.DMA,
            pltpu.SemaphoreType.DMA,
            pltpu.SemaphoreType.DMA,
            pltpu.SemaphoreType.DMA,
        ],
    )(functools.partial(_gather_diff_body, epw, nchunk))
    return f(state, src, dst)


def _sc_scatter_add(msgs, dst):
    epwc = EPC // NW
    nchunkc = epwc // CH
    kch = len(msgs)

    def body(*refs):
        msg_refs = refs[:kch]
        (dst_hbm, out_hbm, idx0, idx1, rows0, rows1, zbuf, acc,
         is0, is1, im0, im1) = refs[kch:]
        cid = lax.axis_index("c")
        sid = lax.axis_index("s")
        wid = sid * NC + cid
        idx = (idx0, idx1)
        rows = (rows0, rows1)
        isem = (is0, is1)
        msem = (im0, im1)

        def zrow(r, c):
            for j in range(D // 16):
                zbuf[r, pl.ds(j * 16, 16)] = jnp.zeros((16,), jnp.float32)
            return c
        lax.fori_loop(0, ZCH, zrow, 0)
        for k in range(KMAX):
            cidx = sid + NS * k

            @pl.when(cidx < NZCH)
            def _():
                pltpu.sync_copy(zbuf, acc.at[pl.ds(cidx * ZCH, ZCH)])
        plsc.subcore_barrier()

        for c in range(kch):
            msg_ref = msg_refs[c]
            dbase = c * EPC + wid * epwc
            mbase = wid * epwc

            def prefetch(j, b):
                pltpu.async_copy(
                    dst_hbm.at[pl.ds(dbase + j * CH, CH)], idx[b], isem[b])
                pltpu.async_copy(
                    msg_ref.at[pl.ds(mbase + j * CH, CH)], rows[b], msem[b])

            prefetch(0, 0)
            prefetch(1, 1)

            def pair(g, cc):
                for b in range(2):
                    j = 2 * g + b

                    @pl.when(j < nchunkc)
                    def _():
                        pltpu.make_async_copy(
                            dst_hbm.at[pl.ds(dbase + j * CH, CH)],
                            idx[b], isem[b]).wait()
                        pltpu.make_async_copy(
                            msg_ref.at[pl.ds(mbase + j * CH, CH)],
                            rows[b], msem[b]).wait()
                        pltpu.sync_copy(rows[b], acc.at[idx[b]], add=True)

                        @pl.when(j + 2 < nchunkc)
                        def _():
                            prefetch(j + 2, b)
                return cc

            lax.fori_loop(0, (nchunkc + 1) // 2, pair, 0)
        plsc.subcore_barrier()

        for k in range(KMAX):
            cidx = sid + NS * k

            @pl.when(cidx < NZCH)
            def _():
                r0 = cidx * ZCH
                pltpu.sync_copy(acc.at[pl.ds(r0, ZCH)],
                                out_hbm.at[cid, pl.ds(r0, ZCH)])

    mesh = plsc.VectorSubcoreMesh(core_axis_name="c", subcore_axis_name="s")
    f = functools.partial(
        pl.kernel,
        out_type=jax.ShapeDtypeStruct((NC, N, D), jnp.float32),
        mesh=mesh,
        scratch_types=[
            pltpu.VMEM((CH,), jnp.int32),
            pltpu.VMEM((CH,), jnp.int32),
            pltpu.VMEM((CH, D), jnp.float32),
            pltpu.VMEM((CH, D), jnp.float32),
            pltpu.VMEM((ZCH, D), jnp.float32),
            pltpu.VMEM_SHARED((N, D), jnp.float32),
            pltpu.SemaphoreType.DMA,
            pltpu.SemaphoreType.DMA,
            pltpu.SemaphoreType.DMA,
            pltpu.SemaphoreType.DMA,
        ],
    )(body)
    return f(*msgs, dst)


_DOT = functools.partial(
    lax.dot_general, precision=lax.Precision.DEFAULT,
    preferred_element_type=jnp.float32)


def _dot_t(x, w):
    return _DOT(x, w, (((1,), (1,)), ((), ())))


def _mlp_body(diff_ref, ef_ref, w1d, w1e, b1, w2, b2,
              a1d, a1e, ba1, a2, ba2, out_ref):
    x = diff_ref[...]
    ef = ef_ref[...]
    h1 = jnp.maximum(_dot_t(x, w1d[...]) + _dot_t(ef, w1e[...]) + b1[...], 0.0)
    msg = _dot_t(h1, w2[...]) + b2[...]
    a1 = jnp.maximum(_dot_t(x, a1d[...]) + _dot_t(ef, a1e[...]) + ba1[...], 0.0)
    att = jax.nn.sigmoid(_dot_t(a1, a2[...]) + ba2[...])
    out_ref[...] = msg * att


def _tc_edge_mlp(diff, ef, W1, b1, W2, b2, A1, ba1, A2, ba2):
    ne = diff.shape[0]
    grid = (ne // BE,)
    w1d, w1e = W1[:, :D], W1[:, D:]
    a1d, a1e = A1[:, :D], A1[:, D:]
    full = lambda s: pl.BlockSpec(s, lambda i: (0, 0))
    return pl.pallas_call(
        _mlp_body,
        grid=grid,
        in_specs=[
            pl.BlockSpec((BE, D), lambda i: (i, 0)),
            pl.BlockSpec((BE, DE), lambda i: (i, 0)),
            full((D, D)), full((D, DE)), full((1, D)),
            full((D, D)), full((1, D)),
            full((D, D)), full((D, DE)), full((1, D)),
            full((D, D)), full((1, D)),
        ],
        out_specs=pl.BlockSpec((BE, D), lambda i: (i, 0)),
        out_shape=jax.ShapeDtypeStruct((ne, D), jnp.float32),
    )(diff, ef, w1d, w1e, b1.reshape(1, D), W2, b2.reshape(1, D),
      a1d, a1e, ba1.reshape(1, D), A2, ba2.reshape(1, D))


def _gru_body(p0_ref, p1_ref, h_ref, wih, bih, whh, bhh, out_ref):
    x = p0_ref[0] + p1_ref[0]
    h = h_ref[...]
    gi = _dot_t(x, wih[...]) + bih[...]
    gh = _dot_t(h, whh[...]) + bhh[...]
    r = jax.nn.sigmoid(gi[:, :D] + gh[:, :D])
    z = jax.nn.sigmoid(gi[:, D:2 * D] + gh[:, D:2 * D])
    n = jnp.tanh(gi[:, 2 * D:] + r * gh[:, 2 * D:])
    out_ref[...] = (1.0 - z) * n + z * h


def _tc_gru(partials, state, W_ih, b_ih, W_hh, b_hh):
    BN = 1000
    grid = (N // BN,)
    full = lambda s: pl.BlockSpec(s, lambda i: (0, 0))
    return pl.pallas_call(
        _gru_body,
        grid=grid,
        in_specs=[
            pl.BlockSpec((1, BN, D), lambda i: (0, i, 0)),
            pl.BlockSpec((1, BN, D), lambda i: (1, i, 0)),
            pl.BlockSpec((BN, D), lambda i: (i, 0)),
            full((3 * D, D)), full((1, 3 * D)),
            full((3 * D, D)), full((1, 3 * D)),
        ],
        out_specs=pl.BlockSpec((BN, D), lambda i: (i, 0)),
        out_shape=jax.ShapeDtypeStruct((N, D), jnp.float32),
    )(partials, partials, state, W_ih, b_ih.reshape(1, 3 * D),
      W_hh, b_hh.reshape(1, 3 * D))


def kernel(node_feat, edge, edge_feat, W1, b1, W2, b2, A1, ba1, A2, ba2,
           W_ih, b_ih, W_hh, b_hh):
    src = edge[:, 0].astype(jnp.int32)
    dst = edge[:, 1].astype(jnp.int32)
    msgs = []
    for c in range(KCH):
        sl = slice(c * EPC, (c + 1) * EPC)
        diff_c = _sc_gather_diff(node_feat, src[sl], dst[sl], EPC)
        msgs.append(_tc_edge_mlp(diff_c, edge_feat[sl],
                                 W1, b1, W2, b2, A1, ba1, A2, ba2))
    partials = _sc_scatter_add(msgs, dst)
    return _tc_gru(partials, node_feat, W_ih, b_ih, W_hh, b_hh)


# speedup vs baseline: 1.0286x; 1.0286x over previous
import functools

import numpy as np

import jax
import jax.numpy as jnp
from jax import lax
from jax.experimental import pallas as pl
from jax.experimental.pallas import tpu as pltpu
from jax.experimental.pallas import tpu_sc as plsc

N = 10000
E = 320000
D = 128
DE = 16

NC = 2
NS = 16
NW = NC * NS

KCH = 5
EPC = E // KCH
CH = 80

ZCH = 80
NZCH = N // ZCH
KMAX = (NZCH + NS - 1) // NS

XXX


def _gather_diff_body(epw, nchunk,
                      state_hbm, src_hbm, dst_hbm, diff_hbm,
                      sidx_all, didx_all,
                      srows0, srows1, drows0, drows1, dbuf0, dbuf1,
                      gs0, gs1, gd0, gd1, wb0, wb1):
    wid = lax.axis_index("s") * NC + lax.axis_index("c")
    base = wid * epw
    srows = (srows0, srows1)
    drows = (drows0, drows1)
    dbuf = (dbuf0, dbuf1)
    gs = (gs0, gs1)
    gd = (gd0, gd1)
    wb = (wb0, wb1)

    pltpu.sync_copy(src_hbm.at[pl.ds(base, epw)], sidx_all)
    pltpu.sync_copy(dst_hbm.at[pl.ds(base, epw)], didx_all)

    def issue_gather(j, b):
        pltpu.async_copy(
            state_hbm.at[sidx_all.at[pl.ds(j * CH, CH)]], srows[b], gs[b])
        pltpu.async_copy(
            state_hbm.at[didx_all.at[pl.ds(j * CH, CH)]], drows[b], gd[b])

    issue_gather(0, 0)
    issue_gather(1, 1)

    def pair(g, c):
        for b in range(2):
            j = 2 * g + b

            @pl.when(j < nchunk)
            def _():
                off = base + j * CH
                pltpu.make_async_copy(
                    state_hbm.at[sidx_all.at[pl.ds(j * CH, CH)]],
                    srows[b], gs[b]).wait()
                pltpu.make_async_copy(
                    state_hbm.at[didx_all.at[pl.ds(j * CH, CH)]],
                    drows[b], gd[b]).wait()

                @pl.when(j >= 2)
                def _():
                    pltpu.make_async_copy(
                        dbuf[b], diff_hbm.at[pl.ds(off, CH)], wb[b]).wait()

                @plsc.parallel_loop(0, CH, 1, unroll=4)
                def _(r):
                    for col in range(D // 16):
                        sl = (r, pl.ds(col * 16, 16))
                        dbuf[b][sl] = srows[b][sl] - drows[b][sl]

                pltpu.async_copy(dbuf[b], diff_hbm.at[pl.ds(off, CH)], wb[b])

                @pl.when(j + 2 < nchunk)
                def _():
                    issue_gather(j + 2, b)
        return c

    lax.fori_loop(0, (nchunk + 1) // 2, pair, 0)
    for b in range(2):
        pltpu.make_async_copy(
            dbuf[b], diff_hbm.at[pl.ds(base, CH)], wb[b]).wait()


def _sc_gather_diff(state, src, dst, ne):
    epw = ne // NW
    nchunk = epw // CH
    mesh = plsc.VectorSubcoreMesh(core_axis_name="c", subcore_axis_name="s")
    f = functools.partial(
        pl.kernel,
        out_type=jax.ShapeDtypeStruct((ne, D), jnp.float32),
        mesh=mesh,
        scratch_types=[
            pltpu.VMEM((epw,), jnp.int32),
            pltpu.VMEM((epw,), jnp.int32),
            pltpu.VMEM((CH, D), jnp.float32),
            pltpu.VMEM((CH, D), jnp.float32),
            pltpu.VMEM((CH, D), jnp.float32),
            pltpu.VMEM((CH, D), jnp.float32),
            pltpu.VMEM((CH, D), jnp.float32),
            pltpu.VMEM((CH, D), jnp.float32),
            pltpu.SemaphoreType.DMA,
            pltpu.SemaphoreType.DMA,
            pltpu.SemaphoreType.DMA,
            pltpu.SemaphoreType.DMA,
            pltpu.SemaphoreType.DMA,
            pltpu.SemaphoreType.DMA,
        ],
    )(functools.partial(_gather_diff_body, epw, nchunk))
    return f(state, src, dst)


def _sc_scatter_add(msgs, dst):
    epwc = EPC // NW
    nchunkc = epwc // CH
    kch = len(msgs)

    def body(*refs):
        msg_refs = refs[:kch]
        (dst_hbm, out_hbm, idx0, idx1, rows0, rows1, zbuf, acc,
         is0, is1, im0, im1) = refs[kch:]
        cid = lax.axis_index("c")
        sid = lax.axis_index("s")
        wid = sid * NC + cid
        idx = (idx0, idx1)
        rows = (rows0, rows1)
        isem = (is0, is1)
        msem = (im0, im1)

        def zrow(r, c):
            for j in range(D // 16):
                zbuf[r, pl.ds(j * 16, 16)] = jnp.zeros((16,), jnp.float32)
            return c
        lax.fori_loop(0, ZCH, zrow, 0)
        for k in range(KMAX):
            cidx = sid + NS * k

            @pl.when(cidx < NZCH)
            def _():
                pltpu.sync_copy(zbuf, acc.at[pl.ds(cidx * ZCH, ZCH)])
        plsc.subcore_barrier()

        for c in range(kch):
            msg_ref = msg_refs[c]
            dbase = c * EPC + wid * epwc
            mbase = wid * epwc

            def prefetch(j, b):
                pltpu.async_copy(
                    dst_hbm.at[pl.ds(dbase + j * CH, CH)], idx[b], isem[b])
                pltpu.async_copy(
                    msg_ref.at[pl.ds(mbase + j * CH, CH)], rows[b], msem[b])

            prefetch(0, 0)
            prefetch(1, 1)

            def pair(g, cc):
                for b in range(2):
                    j = 2 * g + b

                    @pl.when(j < nchunkc)
                    def _():
                        pltpu.make_async_copy(
                            dst_hbm.at[pl.ds(dbase + j * CH, CH)],
                            idx[b], isem[b]).wait()
                        pltpu.make_async_copy(
                            msg_ref.at[pl.ds(mbase + j * CH, CH)],
                            rows[b], msem[b]).wait()
                        pltpu.sync_copy(rows[b], acc.at[idx[b]], add=True)

                        @pl.when(j + 2 < nchunkc)
                        def _():
                            prefetch(j + 2, b)
                return cc

            lax.fori_loop(0, (nchunkc + 1) // 2, pair, 0)
        plsc.subcore_barrier()

        for k in range(KMAX):
            cidx = sid + NS * k

            @pl.when(cidx < NZCH)
            def _():
                r0 = cidx * ZCH
                pltpu.sync_copy(acc.at[pl.ds(r0, ZCH)],
                                out_hbm.at[cid, pl.ds(r0, ZCH)])

    mesh = plsc.VectorSubcoreMesh(core_axis_name="c", subcore_axis_name="s")
    f = functools.partial(
        pl.kernel,
        out_type=jax.ShapeDtypeStruct((NC, N, D), jnp.float32),
        mesh=mesh,
        scratch_types=[
            pltpu.VMEM((CH,), jnp.int32),
            pltpu.VMEM((CH,), jnp.int32),
            pltpu.VMEM((CH, D), jnp.float32),
            pltpu.VMEM((CH, D), jnp.float32),
            pltpu.VMEM((ZCH, D), jnp.float32),
            pltpu.VMEM_SHARED((N, D), jnp.float32),
            pltpu.SemaphoreType.DMA,
            pltpu.SemaphoreType.DMA,
            pltpu.SemaphoreType.DMA,
            pltpu.SemaphoreType.DMA,
        ],
    )(body)
    return f(*msgs, dst)


_DOT = functools.partial(
    lax.dot_general, precision=lax.Precision.DEFAULT,
    preferred_element_type=jnp.float32)


def _dot_t(x, w):
    return _DOT(x, w, (((1,), (1,)), ((), ())))


def _mlp_body(diff_ref, ef_ref, w1d, w1e, b1, w2, b2,
              a1d, a1e, ba1, a2, ba2, out_ref):
    x = diff_ref[...]
    ef = ef_ref[...]
    h1 = jnp.maximum(_dot_t(x, w1d[...]) + _dot_t(ef, w1e[...]) + b1[...], 0.0)
    msg = _dot_t(h1, w2[...]) + b2[...]
    a1 = jnp.maximum(_dot_t(x, a1d[...]) + _dot_t(ef, a1e[...]) + ba1[...], 0.0)
    att = jax.nn.sigmoid(_dot_t(a1, a2[...]) + ba2[...])
    out_ref[...] = msg * att


def _tc_edge_mlp(diff, ef, W1, b1, W2, b2, A1, ba1, A2, ba2):
    ne = diff.shape[0]
    grid = (ne // BE,)
    w1d, w1e = W1[:, :D], W1[:, D:]
    a1d, a1e = A1[:, :D], A1[:, D:]
    full = lambda s: pl.BlockSpec(s, lambda i: (0, 0))
    return pl.pallas_call(
        _mlp_body,
        grid=grid,
        in_specs=[
            pl.BlockSpec((BE, D), lambda i: (i, 0)),
            pl.BlockSpec((BE, DE), lambda i: (i, 0)),
            full((D, D)), full((D, DE)), full((1, D)),
            full((D, D)), full((1, D)),
            full((D, D)), full((D, DE)), full((1, D)),
            full((D, D)), full((1, D)),
        ],
        out_specs=pl.BlockSpec((BE, D), lambda i: (i, 0)),
        out_shape=jax.ShapeDtypeStruct((ne, D), jnp.float32),
    )(diff, ef, w1d, w1e, b1.reshape(1, D), W2, b2.reshape(1, D),
      a1d, a1e, ba1.reshape(1, D), A2, ba2.reshape(1, D))


def _gru_body(p0_ref, p1_ref, h_ref, wih, bih, whh, bhh, out_ref):
    x = p0_ref[0] + p1_ref[0]
    h = h_ref[...]
    gi = _dot_t(x, wih[...]) + bih[...]
    gh = _dot_t(h, whh[...]) + bhh[...]
    r = jax.nn.sigmoid(gi[:, :D] + gh[:, :D])
    z = jax.nn.sigmoid(gi[:, D:2 * D] + gh[:, D:2 * D])
    n = jnp.tanh(gi[:, 2 * D:] + r * gh[:, 2 * D:])
    out_ref[...] = (1.0 - z) * n + z * h


def _tc_gru(partials, state, W_ih, b_ih, W_hh, b_hh):
    BN = 1000
    grid = (N // BN,)
    full = lambda s: pl.BlockSpec(s, lambda i: (0, 0))
    return pl.pallas_call(
        _gru_body,
        grid=grid,
        in_specs=[
            pl.BlockSpec((1, BN, D), lambda i: (0, i, 0)),
            pl.BlockSpec((1, BN, D), lambda i: (1, i, 0)),
            pl.BlockSpec((BN, D), lambda i: (i, 0)),
            full((3 * D, D)), full((1, 3 * D)),
            full((3 * D, D)), full((1, 3 * D)),
        ],
        out_specs=pl.BlockSpec((BN, D), lambda i: (i, 0)),
        out_shape=jax.ShapeDtypeStruct((N, D), jnp.float32),
    )(partials, partials, state, W_ih, b_ih.reshape(1, 3 * D),
      W_hh, b_hh.reshape(1, 3 * D))


def kernel(node_feat, edge, edge_feat, W1, b1, W2, b2, A1, ba1, A2, ba2,
           W_ih, b_ih, W_hh, b_hh):
    src = edge[:, 0].astype(jnp.int32)
    dst = edge[:, 1].astype(jnp.int32)
    msgs = []
    for c in range(KCH):
        sl = slice(c * EPC, (c + 1) * EPC)
        diff_c = _sc_gather_diff(node_feat, src[sl], dst[sl], EPC)
        msgs.append(_tc_edge_mlp(diff_c, edge_feat[sl],
                                 W1, b1, W2, b2, A1, ba1, A2, ba2))
    partials = _sc_scatter_add(msgs, dst)
    return _tc_gru(partials, node_feat, W_ih, b_ih, W_hh, b_hh)


# speedup vs baseline: 1.1171x; 1.0860x over previous
import functools

import jax
import jax.numpy as jnp
from jax import lax
from jax.experimental import pallas as pl
from jax.experimental.pallas import tpu as pltpu
from jax.experimental.pallas import tpu_sc as plsc

N = 10000
E = 320000
D = 128
DE = 16

NC = 2
NS = 16
NW = NC * NS

KCH = 5
EPC = E // KCH
CH = 80
NB = 4

ZCH = 80
NZCH = N // ZCH
KMAX = (NZCH + NS - 1) // NS

BE = 8000


def _gather_diff_body(epw, nchunk,
                      state_hbm, src_hbm, dst_hbm, diff_hbm,
                      sidx_all, didx_all,
                      srows0, srows1, srows2, srows3,
                      drows0, drows1, drows2, drows3,
                      dbuf0, dbuf1, dbuf2, dbuf3,
                      gs0, gs1, gs2, gs3, gd0, gd1, gd2, gd3,
                      wb0, wb1, wb2, wb3):
    wid = lax.axis_index("s") * NC + lax.axis_index("c")
    base = wid * epw
    srows = (srows0, srows1, srows2, srows3)
    drows = (drows0, drows1, drows2, drows3)
    dbuf = (dbuf0, dbuf1, dbuf2, dbuf3)
    gs = (gs0, gs1, gs2, gs3)
    gd = (gd0, gd1, gd2, gd3)
    wb = (wb0, wb1, wb2, wb3)

    pltpu.sync_copy(src_hbm.at[pl.ds(base, epw)], sidx_all)
    pltpu.sync_copy(dst_hbm.at[pl.ds(base, epw)], didx_all)

    def issue_gather(j, b):
        pltpu.async_copy(
            state_hbm.at[sidx_all.at[pl.ds(j * CH, CH)]], srows[b], gs[b])
        pltpu.async_copy(
            state_hbm.at[didx_all.at[pl.ds(j * CH, CH)]], drows[b], gd[b])

    for b in range(NB):
        issue_gather(b, b)

    def pair(g, c):
        for b in range(NB):
            j = NB * g + b

            @pl.when(j < nchunk)
            def _():
                off = base + j * CH
                pltpu.make_async_copy(
                    state_hbm.at[sidx_all.at[pl.ds(j * CH, CH)]],
                    srows[b], gs[b]).wait()
                pltpu.make_async_copy(
                    state_hbm.at[didx_all.at[pl.ds(j * CH, CH)]],
                    drows[b], gd[b]).wait()

                @pl.when(j >= NB)
                def _():
                    pltpu.make_async_copy(
                        dbuf[b], diff_hbm.at[pl.ds(off, CH)], wb[b]).wait()

                @plsc.parallel_loop(0, CH, 1, unroll=4)
                def _(r):
                    for g in range(D // 32):
                        lo = (r, pl.ds(g * 16, 16))
                        hi = (r, pl.ds(64 + g * 16, 16))
                        va = srows[b][lo] - drows[b][lo]
                        vb = srows[b][hi] - drows[b][hi]
                        ai = lax.bitcast_convert_type(va, jnp.int32)
                        bi = lax.bitcast_convert_type(vb, jnp.int32)
                        ar = ai + 0x7FFF + ((ai >> 16) & 1)
                        br = bi + 0x7FFF + ((bi >> 16) & 1)
                        dbuf[b][r, pl.ds(g * 16, 16)] = (
                            ((ar >> 16) & 0xFFFF) | (br & -65536))

                pltpu.async_copy(dbuf[b], diff_hbm.at[pl.ds(off, CH)], wb[b])

                @pl.when(j + NB < nchunk)
                def _():
                    issue_gather(j + NB, b)
        return c

    lax.fori_loop(0, (nchunk + NB - 1) // NB, pair, 0)
    for b in range(NB):
        pltpu.make_async_copy(
            dbuf[b], diff_hbm.at[pl.ds(base, CH)], wb[b]).wait()


def _sc_gather_diff(state, src, dst, ne):
    epw = ne // NW
    nchunk = epw // CH
    mesh = plsc.VectorSubcoreMesh(core_axis_name="c", subcore_axis_name="s")
    f = functools.partial(
        pl.kernel,
        out_type=jax.ShapeDtypeStruct((ne, D // 2), jnp.int32),
        mesh=mesh,
        scratch_types=(
            [pltpu.VMEM((epw,), jnp.int32)] * 2
            + [pltpu.VMEM((CH, D), jnp.float32)] * (2 * NB)
            + [pltpu.VMEM((CH, D // 2), jnp.int32)] * NB
            + [pltpu.SemaphoreType.DMA] * (3 * NB)
        ),
    )(functools.partial(_gather_diff_body, epw, nchunk))
    return f(state, src, dst)


def _sc_scatter_add(msgs, dst, chunk_ids):
    epwc = EPC // NW
    nchunkc = epwc // CH
    kch = len(msgs)

    def body(*refs):
        msg_refs = refs[:kch]
        (dst_hbm, out_hbm, idx0, idx1, rows0, rows1, zbuf, acc,
         is0, is1, im0, im1) = refs[kch:]
        cid = lax.axis_index("c")
        sid = lax.axis_index("s")
        wid = sid * NC + cid
        idx = (idx0, idx1)
        rows = (rows0, rows1)
        isem = (is0, is1)
        msem = (im0, im1)

        def zrow(r, c):
            for j in range(D // 16):
                zbuf[r, pl.ds(j * 16, 16)] = jnp.zeros((16,), jnp.float32)
            return c
        lax.fori_loop(0, ZCH, zrow, 0)
        for k in range(KMAX):
            cidx = sid + NS * k

            @pl.when(cidx < NZCH)
            def _():
                pltpu.sync_copy(zbuf, acc.at[pl.ds(cidx * ZCH, ZCH)])
        plsc.subcore_barrier()

        for c in range(kch):
            msg_ref = msg_refs[c]
            dbase = chunk_ids[c] * EPC + wid * epwc
            mbase = wid * epwc

            def prefetch(j, b):
                pltpu.async_copy(
                    dst_hbm.at[pl.ds(dbase + j * CH, CH)], idx[b], isem[b])
                pltpu.async_copy(
                    msg_ref.at[pl.ds(mbase + j * CH, CH)], rows[b], msem[b])

            prefetch(0, 0)
            prefetch(1, 1)

            def pair(g, cc):
                for b in range(2):
                    j = 2 * g + b

                    @pl.when(j < nchunkc)
                    def _():
                        pltpu.make_async_copy(
                            dst_hbm.at[pl.ds(dbase + j * CH, CH)],
                            idx[b], isem[b]).wait()
                        pltpu.make_async_copy(
                            msg_ref.at[pl.ds(mbase + j * CH, CH)],
                            rows[b], msem[b]).wait()
                        pltpu.sync_copy(rows[b], acc.at[idx[b]], add=True)

                        @pl.when(j + 2 < nchunkc)
                        def _():
                            prefetch(j + 2, b)
                return cc

            lax.fori_loop(0, (nchunkc + 1) // 2, pair, 0)
        plsc.subcore_barrier()

        for k in range(KMAX):
            cidx = sid + NS * k

            @pl.when(cidx < NZCH)
            def _():
                r0 = cidx * ZCH
                pltpu.sync_copy(acc.at[pl.ds(r0, ZCH)],
                                out_hbm.at[cid, pl.ds(r0, ZCH)])

    mesh = plsc.VectorSubcoreMesh(core_axis_name="c", subcore_axis_name="s")
    f = functools.partial(
        pl.kernel,
        out_type=jax.ShapeDtypeStruct((NC, N, D), jnp.float32),
        mesh=mesh,
        scratch_types=[
            pltpu.VMEM((CH,), jnp.int32),
            pltpu.VMEM((CH,), jnp.int32),
            pltpu.VMEM((CH, D), jnp.float32),
            pltpu.VMEM((CH, D), jnp.float32),
            pltpu.VMEM((ZCH, D), jnp.float32),
            pltpu.VMEM_SHARED((N, D), jnp.float32),
            pltpu.SemaphoreType.DMA,
            pltpu.SemaphoreType.DMA,
            pltpu.SemaphoreType.DMA,
            pltpu.SemaphoreType.DMA,
        ],
    )(body)
    return f(*msgs, dst)


_DOT = functools.partial(
    lax.dot_general, precision=lax.Precision.DEFAULT,
    preferred_element_type=jnp.float32)


def _dot_t(x, w):
    return _DOT(x, w, (((1,), (1,)), ((), ())))


def _mlp_body(diff_ref, ef_ref, w1d, w1e, b1, w2, b2,
              a1d, a1e, ba1, a2, ba2, out_ref):
    w = diff_ref[...]
    xa = lax.bitcast_convert_type(w << 16, jnp.float32)
    xb = lax.bitcast_convert_type(w & -65536, jnp.float32)
    x = jnp.concatenate([xa, xb], axis=1)
    ef = ef_ref[...]
    h1 = jnp.maximum(_dot_t(x, w1d[...]) + _dot_t(ef, w1e[...]) + b1[...], 0.0)
    msg = _dot_t(h1, w2[...]) + b2[...]
    a1 = jnp.maximum(_dot_t(x, a1d[...]) + _dot_t(ef, a1e[...]) + ba1[...], 0.0)
    att = jax.nn.sigmoid(_dot_t(a1, a2[...]) + ba2[...])
    out_ref[...] = msg * att


def _tc_edge_mlp(diff, ef, W1, b1, W2, b2, A1, ba1, A2, ba2):
    ne = diff.shape[0]
    grid = (ne // BE,)
    w1d, w1e = W1[:, :D], W1[:, D:]
    a1d, a1e = A1[:, :D], A1[:, D:]
    full = lambda s: pl.BlockSpec(s, lambda i: (0, 0))
    return pl.pallas_call(
        _mlp_body,
        grid=grid,
        compiler_params=pltpu.CompilerParams(
            dimension_semantics=("parallel",)),
        in_specs=[
            pl.BlockSpec((BE, D // 2), lambda i: (i, 0)),
            pl.BlockSpec((BE, DE), lambda i: (i, 0)),
            full((D, D)), full((D, DE)), full((1, D)),
            full((D, D)), full((1, D)),
            full((D, D)), full((D, DE)), full((1, D)),
            full((D, D)), full((1, D)),
        ],
        out_specs=pl.BlockSpec((BE, D), lambda i: (i, 0)),
        out_shape=jax.ShapeDtypeStruct((ne, D), jnp.float32),
    )(diff, ef, w1d, w1e, b1.reshape(1, D), W2, b2.reshape(1, D),
      a1d, a1e, ba1.reshape(1, D), A2, ba2.reshape(1, D))


def _gru_body(a0_ref, a1_ref, b0_ref, b1_ref, h_ref, wih, bih, whh, bhh,
              out_ref):
    x = a0_ref[0] + a1_ref[0] + b0_ref[0] + b1_ref[0]
    h = h_ref[...]
    gi = _dot_t(x, wih[...]) + bih[...]
    gh = _dot_t(h, whh[...]) + bhh[...]
    r = jax.nn.sigmoid(gi[:, :D] + gh[:, :D])
    z = jax.nn.sigmoid(gi[:, D:2 * D] + gh[:, D:2 * D])
    n = jnp.tanh(gi[:, 2 * D:] + r * gh[:, 2 * D:])
    out_ref[...] = (1.0 - z) * n + z * h


def _tc_gru(pa, pb, state, W_ih, b_ih, W_hh, b_hh):
    BN = 1000
    grid = (N // BN,)
    full = lambda s: pl.BlockSpec(s, lambda i: (0, 0))
    return pl.pallas_call(
        _gru_body,
        grid=grid,
        compiler_params=pltpu.CompilerParams(
            dimension_semantics=("parallel",)),
        in_specs=[
            pl.BlockSpec((1, BN, D), lambda i: (0, i, 0)),
            pl.BlockSpec((1, BN, D), lambda i: (1, i, 0)),
            pl.BlockSpec((1, BN, D), lambda i: (0, i, 0)),
            pl.BlockSpec((1, BN, D), lambda i: (1, i, 0)),
            pl.BlockSpec((BN, D), lambda i: (i, 0)),
            full((3 * D, D)), full((1, 3 * D)),
            full((3 * D, D)), full((1, 3 * D)),
        ],
        out_specs=pl.BlockSpec((BN, D), lambda i: (i, 0)),
        out_shape=jax.ShapeDtypeStruct((N, D), jnp.float32),
    )(pa, pa, pb, pb, state, W_ih, b_ih.reshape(1, 3 * D),
      W_hh, b_hh.reshape(1, 3 * D))


def kernel(node_feat, edge, edge_feat, W1, b1, W2, b2, A1, ba1, A2, ba2,
           W_ih, b_ih, W_hh, b_hh):
    src = edge[:, 0].astype(jnp.int32)
    dst = edge[:, 1].astype(jnp.int32)
    diffs = []
    for c in range(KCH):
        sl = slice(c * EPC, (c + 1) * EPC)
        diffs.append(_sc_gather_diff(node_feat, src[sl], dst[sl], EPC))
    msgs = []
    for c in range(KCH):
        sl = slice(c * EPC, (c + 1) * EPC)
        msgs.append(_tc_edge_mlp(diffs[c], edge_feat[sl],
                                 W1, b1, W2, b2, A1, ba1, A2, ba2))
    pa = _sc_scatter_add(msgs[:3], dst, (0, 1, 2))
    pb = _sc_scatter_add(msgs[3:], dst, (3, 4))
    return _tc_gru(pa, pb, node_feat, W_ih, b_ih, W_hh, b_hh)
